# Initial kernel scaffold; baseline (speedup 1.0000x reference)
import jax
import jax.numpy as jnp
from jax.experimental import pallas as pl


def kernel(user_code, item_code, user_emb, item_emb):
    raise NotImplementedError("write your pallas kernel here")


# speedup vs baseline: 1.2963x; 1.2963x over previous
import functools

import jax
import jax.numpy as jnp
from jax import lax
from jax.experimental import pallas as pl
from jax.experimental.pallas import tpu as pltpu
from jax.experimental.pallas import tpu_sc as plsc

B = 16384
NL = 50
D = 32
NC = 2
NS = 16
NW = NC * NS
USERS_PER_W = B // NW
BLOCK_USERS = 32
N_BLOCKS = USERS_PER_W // BLOCK_USERS
ROWS_PER_BLOCK = BLOCK_USERS * NL
CHUNK = 128
N_CHUNKS = (ROWS_PER_BLOCK + CHUNK - 1) // CHUNK


def _body(uc_hbm, ic_hbm, ue_hbm, ie_hbm, out_hbm,
          idx_v, uidx_v, rows_v, u_v, out_v, sem_r, sem_u):
    wid = lax.axis_index("s") * NC + lax.axis_index("c")
    base_user = wid * USERS_PER_W
    lane = lax.iota(jnp.int32, 16)

    def gather_chunks(idxv, rowsv):
        copies = []
        for k in range(N_CHUNKS):
            start = k * CHUNK
            ln = min(CHUNK, ROWS_PER_BLOCK - start)
            copies.append((ie_hbm.at[idxv.at[pl.ds(start, ln)]],
                           rowsv.at[pl.ds(start, ln)]))
        return copies

    @pl.loop(0, N_BLOCKS)
    def block_loop(g):
        boff = base_user + g * BLOCK_USERS
        pltpu.sync_copy(ic_hbm.at[pl.ds(boff * NL, ROWS_PER_BLOCK)], idx_v)
        pltpu.sync_copy(uc_hbm.at[pl.ds(boff, BLOCK_USERS)], uidx_v)
        for src, dst in gather_chunks(idx_v, rows_v):
            pltpu.async_copy(src, dst, sem_r)
        pltpu.async_copy(ue_hbm.at[uidx_v], u_v, sem_u)
        for src, dst in gather_chunks(idx_v, rows_v):
            pltpu.make_async_copy(src, dst, sem_r).wait()
        pltpu.make_async_copy(ue_hbm.at[uidx_v], u_v, sem_u).wait()

        for s in range(BLOCK_USERS // 16):
            urow = lane + (16 * s)
            ucols = [
                plsc.load_gather(u_v, [urow, jnp.full((16,), d, jnp.int32)])
                for d in range(D)
            ]

            @plsc.parallel_loop(0, NL)
            def lbody(l, _urow=urow, _ucols=ucols, _s=s):
                row = lane * NL + (800 * _s + l)
                acc = jnp.zeros((16,), jnp.float32)
                for d in range(D):
                    col = plsc.load_gather(
                        rows_v, [row, jnp.full((16,), d, jnp.int32)])
                    acc = acc + col * _ucols[d]
                plsc.store_scatter(
                    out_v, [_urow, jnp.zeros((16,), jnp.int32) + l], acc)

        pltpu.sync_copy(out_v, out_hbm.at[pl.ds(boff, BLOCK_USERS)])


@jax.jit
def _mf(user_code, item_code_flat, user_emb, item_emb):
    f = pl.kernel(
        _body,
        out_type=jax.ShapeDtypeStruct((B, NL), jnp.float32),
        mesh=plsc.VectorSubcoreMesh(core_axis_name="c", subcore_axis_name="s"),
        compiler_params=pltpu.CompilerParams(
            needs_layout_passes=False, use_tc_tiling_on_sc=False),
        scratch_types=[
            pltpu.VMEM((ROWS_PER_BLOCK,), jnp.int32),
            pltpu.VMEM((BLOCK_USERS,), jnp.int32),
            pltpu.VMEM((ROWS_PER_BLOCK, D), jnp.float32),
            pltpu.VMEM((BLOCK_USERS, D), jnp.float32),
            pltpu.VMEM((BLOCK_USERS, NL), jnp.float32),
            pltpu.SemaphoreType.DMA,
            pltpu.SemaphoreType.DMA,
        ],
    )
    return f(user_code, item_code_flat, user_emb, item_emb)


def kernel(user_code, item_code, user_emb, item_emb):
    return _mf(user_code, item_code.reshape(-1), user_emb, item_emb)
